# baseline (speedup 1.0000x reference)
import jax
import jax.numpy as jnp
from jax.experimental import pallas as pl
from jax.experimental.pallas import tpu as pltpu

_N = 4096
_D = 128
_BM = 1024
_BN = 4096
_SUB = 512
_NSI = _BM // _SUB
_NSJ = _BN // _SUB
_NT = _N // _SUB


def _seg_body(bat_sm_ref, bat_ref, cls_ref, z1_ref, z2_ref, out_ref,
              kr_ref, kc_ref):
    bi = pl.program_id(0)

    @pl.when(bi == 0)
    def _prologue():
        bat = bat_ref[...].reshape(1, _N)
        cls = cls_ref[...].reshape(1, _N)
        valid = (cls != 24) & (cls != 25) & (cls != 26)
        lane = jax.lax.broadcasted_iota(jnp.int32, (1, _N), 1)
        kc = jnp.where(valid, bat * 32 + cls, -lane - 1)
        kc_ref[...] = kc
        kr_ref[...] = kc.reshape(_N, 1)

    for si in range(_NSI):
        for gj in range(_NSJ):
            gi = bi * _NSI + si
            rlo = bat_sm_ref[gi * _SUB]
            rhi = bat_sm_ref[gi * _SUB + _SUB - 1]
            clo = bat_sm_ref[gj * _SUB]
            chi = bat_sm_ref[gj * _SUB + _SUB - 1]
            inter = (rlo <= chi) & (clo <= rhi)
            rs = slice(si * _SUB, (si + 1) * _SUB)
            cs = slice(gj * _SUB, (gj + 1) * _SUB)

            def _masked_prod(si=si, gj=gj):
                a = z1_ref[si * _SUB:(si + 1) * _SUB, :]
                b = z2_ref[gj * _SUB:(gj + 1) * _SUB, :]
                prod = jax.lax.dot_general(
                    a, b, (((1,), (1,)), ((), ())),
                    preferred_element_type=jnp.float32)
                rk = kr_ref[pl.ds((bi * _NSI + si) * _SUB, _SUB), :]
                ck = kc_ref[:, gj * _SUB:(gj + 1) * _SUB]
                return prod, rk == ck

            @pl.when(inter & (gi == gj))
            def _compute_diag(rs=rs, cs=cs, mp=_masked_prod):
                prod, mask = mp()
                rid = jax.lax.broadcasted_iota(jnp.int32, (_SUB, _SUB), 0)
                cid = jax.lax.broadcasted_iota(jnp.int32, (_SUB, _SUB), 1)
                mask = mask & (rid != cid)
                out_ref[rs, cs] = jnp.where(mask, prod, jnp.float32(0.0))

            @pl.when(inter & (gi != gj))
            def _compute_offdiag(rs=rs, cs=cs, mp=_masked_prod):
                prod, mask = mp()
                out_ref[rs, cs] = jnp.where(mask, prod, jnp.float32(0.0))

            @pl.when(jnp.logical_not(inter))
            def _zero(rs=rs, cs=cs):
                out_ref[rs, cs] = jnp.zeros((_SUB, _SUB), jnp.float32)


def kernel(z1, z2, cls_label, batch):
    cls = cls_label.astype(jnp.int32)
    bat = batch.astype(jnp.int32)
    n = cls.shape[0]

    out = pl.pallas_call(
        _seg_body,
        grid=(_N // _BM,),
        in_specs=[
            pl.BlockSpec(memory_space=pltpu.SMEM),
            pl.BlockSpec((_N,), lambda i: (0,)),
            pl.BlockSpec((_N,), lambda i: (0,)),
            pl.BlockSpec((_BM, _D), lambda i: (i, 0)),
            pl.BlockSpec((_N, _D), lambda i: (0, 0)),
        ],
        out_specs=pl.BlockSpec((_BM, _BN), lambda i: (i, 0)),
        out_shape=jax.ShapeDtypeStruct((n, n), jnp.float32),
        scratch_shapes=[
            pltpu.VMEM((_N, 1), jnp.int32),
            pltpu.VMEM((1, _N), jnp.int32),
        ],
        compiler_params=pltpu.CompilerParams(
            dimension_semantics=("arbitrary",)),
    )(bat, bat, cls, z1, z2)
    return out


# speedup vs baseline: 1.0850x; 1.0850x over previous
import jax
import jax.numpy as jnp
from jax.experimental import pallas as pl
from jax.experimental.pallas import tpu as pltpu

_N = 4096
_D = 128
_BM = 512
_BN = 4096
_SUB = 512
_NSI = _BM // _SUB
_NSJ = _BN // _SUB
_NT = _N // _SUB


def _seg_body(bat_sm_ref, bat_ref, cls_ref, z1_ref, z2_ref, out_ref,
              kr_ref, kc_ref):
    bi = pl.program_id(0)

    @pl.when(bi == 0)
    def _prologue():
        bat = bat_ref[...].reshape(1, _N)
        cls = cls_ref[...].reshape(1, _N)
        valid = (cls != 24) & (cls != 25) & (cls != 26)
        lane = jax.lax.broadcasted_iota(jnp.int32, (1, _N), 1)
        kc = jnp.where(valid, bat * 32 + cls, -lane - 1)
        kc_ref[...] = kc
        kr_ref[...] = kc.reshape(_N, 1)

    for si in range(_NSI):
        for gj in range(_NSJ):
            gi = bi * _NSI + si
            rlo = bat_sm_ref[gi * _SUB]
            rhi = bat_sm_ref[gi * _SUB + _SUB - 1]
            clo = bat_sm_ref[gj * _SUB]
            chi = bat_sm_ref[gj * _SUB + _SUB - 1]
            inter = (rlo <= chi) & (clo <= rhi)
            rs = slice(si * _SUB, (si + 1) * _SUB)
            cs = slice(gj * _SUB, (gj + 1) * _SUB)

            def _masked_prod(si=si, gj=gj):
                a = z1_ref[si * _SUB:(si + 1) * _SUB, :]
                b = z2_ref[gj * _SUB:(gj + 1) * _SUB, :]
                prod = jax.lax.dot_general(
                    a, b, (((1,), (1,)), ((), ())),
                    preferred_element_type=jnp.float32)
                rk = kr_ref[pl.ds((bi * _NSI + si) * _SUB, _SUB), :]
                ck = kc_ref[:, gj * _SUB:(gj + 1) * _SUB]
                return prod, rk == ck

            @pl.when(inter & (gi == gj))
            def _compute_diag(rs=rs, cs=cs, mp=_masked_prod):
                prod, mask = mp()
                rid = jax.lax.broadcasted_iota(jnp.int32, (_SUB, _SUB), 0)
                cid = jax.lax.broadcasted_iota(jnp.int32, (_SUB, _SUB), 1)
                mask = mask & (rid != cid)
                out_ref[rs, cs] = jnp.where(mask, prod, jnp.float32(0.0))

            @pl.when(inter & (gi != gj))
            def _compute_offdiag(rs=rs, cs=cs, mp=_masked_prod):
                prod, mask = mp()
                out_ref[rs, cs] = jnp.where(mask, prod, jnp.float32(0.0))

            @pl.when(jnp.logical_not(inter))
            def _zero(rs=rs, cs=cs):
                out_ref[rs, cs] = jnp.zeros((_SUB, _SUB), jnp.float32)


def kernel(z1, z2, cls_label, batch):
    cls = cls_label.astype(jnp.int32)
    bat = batch.astype(jnp.int32)
    n = cls.shape[0]

    out = pl.pallas_call(
        _seg_body,
        grid=(_N // _BM,),
        in_specs=[
            pl.BlockSpec(memory_space=pltpu.SMEM),
            pl.BlockSpec((_N,), lambda i: (0,)),
            pl.BlockSpec((_N,), lambda i: (0,)),
            pl.BlockSpec((_BM, _D), lambda i: (i, 0)),
            pl.BlockSpec((_N, _D), lambda i: (0, 0)),
        ],
        out_specs=pl.BlockSpec((_BM, _BN), lambda i: (i, 0)),
        out_shape=jax.ShapeDtypeStruct((n, n), jnp.float32),
        scratch_shapes=[
            pltpu.VMEM((_N, 1), jnp.int32),
            pltpu.VMEM((1, _N), jnp.int32),
        ],
        compiler_params=pltpu.CompilerParams(
            dimension_semantics=("arbitrary",)),
    )(bat, bat, cls, z1, z2)
    return out
